# baseline (speedup 1.0000x reference)
import jax
import jax.numpy as jnp
from jax.experimental import pallas as pl
from jax.experimental.pallas import tpu as pltpu

DEPTH = 1000
ROWS_PER_BLOCK = 512


def _one_hot_block(idx_ref, out_ref):
    idx = idx_ref[:].astype(jnp.int32)
    iota = jax.lax.broadcasted_iota(jnp.int32, (ROWS_PER_BLOCK, DEPTH), 1)
    out_ref[0] = (iota == idx[:, None]).astype(jnp.float32)


def kernel(x):
    n = x.size
    x_flat = jnp.reshape(x, (n,))
    num_blocks = n // ROWS_PER_BLOCK
    out = pl.pallas_call(
        _one_hot_block,
        grid=(num_blocks,),
        in_specs=[pl.BlockSpec((ROWS_PER_BLOCK,), lambda i: (i,))],
        out_specs=pl.BlockSpec((1, ROWS_PER_BLOCK, DEPTH), lambda i: (0, i, 0)),
        out_shape=jax.ShapeDtypeStruct((1, n, DEPTH), jnp.float32),
    )(x_flat)
    return out
